# Initial kernel scaffold; baseline (speedup 1.0000x reference)
import jax
import jax.numpy as jnp
from jax import lax
from jax.experimental import pallas as pl
from jax.experimental.pallas import tpu as pltpu
from jax.experimental.pallas import tpu_sc as plsc

NC = 2
NS = 16
LANES = 16

N = 10000
E = 320000
D = 128
H = 128
B = 128

CHUNK = 128
NCH = 80
PT = NCH * CHUNK
EPA = NC * NS * PT
NA = 10240
ROWS_PER_TILE = NA // NS

_f32 = jnp.float32
_i32 = jnp.int32


def _agg_body(x_hbm, srcm_hbm, dstm_hbm, parts_hbm, cnts_hbm,
              acc_sp, cnt_sp, zbuf, zcnt, srci, dsti, rows0, rows1,
              ones_v, sem0, sem1):
    c = lax.axis_index("c")
    s = lax.axis_index("s")

    def zrow(i, _):
        for j in range(D // LANES):
            zbuf[i, pl.ds(LANES * j, LANES)] = jnp.zeros((LANES,), _f32)
        return 0
    lax.fori_loop(0, CHUNK, zrow, 0)

    def zc(i, _):
        zcnt[pl.ds(LANES * i, LANES)] = jnp.zeros((LANES,), _f32)
        return 0
    lax.fori_loop(0, ROWS_PER_TILE // LANES, zc, 0)

    for j in range(CHUNK // LANES):
        ones_v[pl.ds(LANES * j, LANES)] = jnp.ones((LANES,), _f32)

    for k in range(ROWS_PER_TILE // CHUNK):
        pltpu.sync_copy(zbuf, acc_sp.at[pl.ds(s * ROWS_PER_TILE + k * CHUNK, CHUNK)])
    pltpu.sync_copy(zcnt, cnt_sp.at[pl.ds(s * ROWS_PER_TILE, ROWS_PER_TILE)])
    plsc.subcore_barrier()

    pltpu.sync_copy(srcm_hbm.at[c, s], srci)
    pltpu.sync_copy(dstm_hbm.at[c, s], dsti)

    pltpu.async_copy(x_hbm.at[srci.at[0]], rows0, sem0)
    pltpu.async_copy(x_hbm.at[srci.at[1]], rows1, sem1)
    rows = (rows0, rows1)
    sems = (sem0, sem1)

    def step(i2, _):
        for b in range(2):
            ci = 2 * i2 + b
            pltpu.make_async_copy(x_hbm.at[srci.at[ci]], rows[b], sems[b]).wait()
            pltpu.sync_copy(rows[b], acc_sp.at[dsti.at[ci]], add=True)
            pltpu.sync_copy(ones_v, cnt_sp.at[dsti.at[ci]], add=True)

            @pl.when(ci + 2 < NCH)
            def _():
                pltpu.async_copy(x_hbm.at[srci.at[ci + 2]], rows[b], sems[b])
        return 0
    lax.fori_loop(0, NCH // 2, step, 0)
    plsc.subcore_barrier()

    off = c * NA + s * ROWS_PER_TILE
    pltpu.sync_copy(acc_sp.at[pl.ds(s * ROWS_PER_TILE, ROWS_PER_TILE)],
                    parts_hbm.at[pl.ds(off, ROWS_PER_TILE)])
    pltpu.sync_copy(cnt_sp.at[pl.ds(s * ROWS_PER_TILE, ROWS_PER_TILE)],
                    cnts_hbm.at[pl.ds(off, ROWS_PER_TILE)])


def _stage_a(x, src_t, dst_t):
    mesh = plsc.VectorSubcoreMesh(core_axis_name="c", subcore_axis_name="s",
                                  num_cores=NC, num_subcores=NS)
    return pl.kernel(
        _agg_body,
        out_type=(jax.ShapeDtypeStruct((NC * NA, D), _f32),
                  jax.ShapeDtypeStruct((NC * NA,), _f32)),
        mesh=mesh,
        scratch_types=[
            pltpu.VMEM_SHARED((NA, D), _f32),
            pltpu.VMEM_SHARED((NA,), _f32),
            pltpu.VMEM((CHUNK, D), _f32),
            pltpu.VMEM((ROWS_PER_TILE,), _f32),
            pltpu.VMEM((NCH, CHUNK), _i32),
            pltpu.VMEM((NCH, CHUNK), _i32),
            pltpu.VMEM((CHUNK, D), _f32),
            pltpu.VMEM((CHUNK, D), _f32),
            pltpu.VMEM((CHUNK,), _f32),
            pltpu.SemaphoreType.DMA,
            pltpu.SemaphoreType.DMA,
        ],
        name="sage_edge_agg",
    )(x, src_t, dst_t)


def _mlp_body(p_ref, c_ref, x_ref, wl_ref, wr_ref, bl_ref, o_ref):
    p = p_ref[...]
    cm = c_ref[...]
    summed = p[0] + p[1]
    cnt = cm[0] + cm[1]
    mean = summed / jnp.maximum(cnt, 1.0)
    dn = (((1,), (1,)), ((), ()))
    h = lax.dot_general(mean, wl_ref[...], dn, preferred_element_type=_f32)
    h = h + lax.dot_general(x_ref[...], wr_ref[...], dn,
                            preferred_element_type=_f32)
    h = h + bl_ref[...]
    o_ref[...] = jnp.maximum(h, 0.0)


def _stage_b(parts, cnts, x_pad, W_l, W_r, b_l):
    RB = 512
    grid = (NA // RB,)
    return pl.pallas_call(
        _mlp_body,
        grid=grid,
        in_specs=[
            pl.BlockSpec((2, RB, D), lambda i: (0, i, 0)),
            pl.BlockSpec((2, RB, 1), lambda i: (0, i, 0)),
            pl.BlockSpec((RB, D), lambda i: (i, 0)),
            pl.BlockSpec((H, D), lambda i: (0, 0)),
            pl.BlockSpec((H, D), lambda i: (0, 0)),
            pl.BlockSpec((1, H), lambda i: (0, 0)),
        ],
        out_specs=pl.BlockSpec((RB, H), lambda i: (i, 0)),
        out_shape=jax.ShapeDtypeStruct((NA, H), _f32),
        name="sage_mlp",
    )(parts.reshape(NC, NA, D), cnts.reshape(NC, NA, 1), x_pad,
      W_l, W_r, b_l.reshape(1, H))


def _pool_body(h_hbm, batch_hbm, pooled_hbm, bbuf, cbuf, acc4, sem):
    del sem
    c = lax.axis_index("c")
    s = lax.axis_index("s")
    wid = c * NS + s
    g0 = wid * (B // (NC * NS))

    BCH = 2000
    carry = tuple(jnp.int32(0) for _ in range(5))
    for t in range(N // BCH):
        pltpu.sync_copy(batch_hbm.at[pl.ds(t * BCH, BCH)], bbuf)

        def cb(j, carry):
            v = bbuf[pl.ds(j * LANES, LANES)]
            outs = []
            for k in range(5):
                m = jnp.where(v < g0 + k, jnp.int32(1), jnp.int32(0))
                outs.append(carry[k] + jnp.sum(m))
            return tuple(outs)
        carry = lax.fori_loop(0, BCH // LANES, cb, carry)

    RCH = 64
    for k in range(4):
        start = carry[k]
        end = carry[k + 1]
        nch = (end - start + (RCH - 1)) // RCH

        def chunk(cix, accs):
            off = start + cix * RCH
            pltpu.sync_copy(h_hbm.at[pl.ds(off, RCH)], cbuf)

            def row(r, accs):
                valid = (off + r) < end
                return tuple(
                    jnp.where(valid,
                              jnp.maximum(accs[j],
                                          cbuf[r, pl.ds(LANES * j, LANES)]),
                              accs[j])
                    for j in range(H // LANES))
            return lax.fori_loop(0, RCH, row, accs)

        accs = lax.fori_loop(0, nch, chunk,
                             tuple(jnp.zeros((LANES,), _f32)
                                   for _ in range(H // LANES)))
        for j in range(H // LANES):
            acc4[k, pl.ds(LANES * j, LANES)] = accs[j]

    pltpu.sync_copy(acc4, pooled_hbm.at[pl.ds(wid * 4, 4)])


def _stage_c(h, batch):
    mesh = plsc.VectorSubcoreMesh(core_axis_name="c", subcore_axis_name="s",
                                  num_cores=NC, num_subcores=NS)
    return pl.kernel(
        _pool_body,
        out_type=jax.ShapeDtypeStruct((B, H), _f32),
        mesh=mesh,
        scratch_types=[
            pltpu.VMEM((2000,), _i32),
            pltpu.VMEM((64, H), _f32),
            pltpu.VMEM((4, H), _f32),
            pltpu.SemaphoreType.DMA,
        ],
        name="sage_pool",
    )(h, batch)


def _head_body(p_ref, w4_ref, b4_ref, o_ref):
    dn = (((1,), (1,)), ((), ()))
    logits = lax.dot_general(p_ref[...], w4_ref[...], dn,
                             preferred_element_type=_f32)
    logits = logits + b4_ref[...]
    m = jnp.max(logits, axis=1, keepdims=True)
    z = logits - m
    lse = jnp.log(jnp.sum(jnp.exp(z), axis=1, keepdims=True))
    o_ref[...] = z - lse


def _stage_d(pooled, W4, b4):
    C = W4.shape[0]
    return pl.pallas_call(
        _head_body,
        out_shape=jax.ShapeDtypeStruct((B, C), _f32),
        name="sage_head",
    )(pooled, W4, b4.reshape(1, C))


@jax.jit
def kernel(x, edge_index, batch, embedding_data, W_l, b_l, W_r, W4, b4):
    del embedding_data

    pad = EPA - E
    pidx = jnp.arange(pad, dtype=_i32)
    src_p = jnp.concatenate([edge_index[0], pidx % N])
    dst_p = jnp.concatenate([edge_index[1], N + pidx % (NA - N)])
    src_t = src_p.reshape(NC, NS, NCH, CHUNK)
    dst_t = dst_p.reshape(NC, NS, NCH, CHUNK)

    x_pad = jnp.concatenate([x, jnp.zeros((NA - N, D), _f32)])

    parts, cnts = _stage_a(x_pad, src_t, dst_t)
    h = _stage_b(parts, cnts, x_pad, W_l, W_r, b_l)
    pooled = _stage_c(h, batch)
    return _stage_d(pooled, W4, b4)


# speedup vs baseline: 10.7720x; 10.7720x over previous
import jax
import jax.numpy as jnp
from jax import lax
from jax.experimental import pallas as pl
from jax.experimental.pallas import tpu as pltpu
from jax.experimental.pallas import tpu_sc as plsc

NC = 2
NS = 16
LANES = 16

N = 10000
E = 320000
D = 128
H = 128
B = 128

CHUNK = 128
NCH = 80
GRP = 16
NGRP = NCH // GRP
PT = NCH * CHUNK
EPA = NC * NS * PT
NA = 10240
ROWS_PER_TILE = NA // NS

_f32 = jnp.float32
_i32 = jnp.int32


def _agg_body(x_hbm, srcm_hbm, dstm_hbm, parts_hbm, cnts_hbm,
              acc_sp, cnt_sp, srci0, srci1, dsti0, dsti1, rows0, rows1,
              ones_v, sem0, sem1, isem0, isem1):
    c = lax.axis_index("c")
    s = lax.axis_index("s")

    def zrow(i, _):
        for j in range(D // LANES):
            rows0[i, pl.ds(LANES * j, LANES)] = jnp.zeros((LANES,), _f32)
        return 0
    lax.fori_loop(0, CHUNK, zrow, 0)

    for j in range(CHUNK // LANES):
        ones_v[pl.ds(LANES * j, LANES)] = jnp.ones((LANES,), _f32)

    for k in range(ROWS_PER_TILE // CHUNK):
        pltpu.sync_copy(rows0, acc_sp.at[pl.ds(s * ROWS_PER_TILE + k * CHUNK, CHUNK)])
        pltpu.sync_copy(rows0.at[0],
                        cnt_sp.at[pl.ds(s * ROWS_PER_TILE + k * CHUNK, CHUNK)])
    plsc.subcore_barrier()

    rows = (rows0, rows1)
    sems = (sem0, sem1)
    ibufs = ((srci0, dsti0, isem0), (srci1, dsti1, isem1))

    pltpu.sync_copy(srcm_hbm.at[c, s, 0], srci0)
    pltpu.sync_copy(dstm_hbm.at[c, s, 0], dsti0)
    pltpu.async_copy(srcm_hbm.at[c, s, 1], srci1, isem1)
    pltpu.async_copy(dstm_hbm.at[c, s, 1], dsti1, isem1)

    for g in range(NGRP):
        sb, db, isem = ibufs[g % 2]
        if g > 0:
            pltpu.make_async_copy(srcm_hbm.at[c, s, g], sb, isem).wait()
            pltpu.make_async_copy(dstm_hbm.at[c, s, g], db, isem).wait()
            if g + 1 < NGRP:
                nsb, ndb, nisem = ibufs[(g + 1) % 2]
                pltpu.async_copy(srcm_hbm.at[c, s, g + 1], nsb, nisem)
                pltpu.async_copy(dstm_hbm.at[c, s, g + 1], ndb, nisem)
        pltpu.async_copy(x_hbm.at[sb.at[0]], rows0, sem0)
        pltpu.async_copy(x_hbm.at[sb.at[1]], rows1, sem1)

        def step(i2, _):
            for b in range(2):
                li = 2 * i2 + b
                pltpu.make_async_copy(x_hbm.at[sb.at[li]], rows[b], sems[b]).wait()
                pltpu.sync_copy(rows[b], acc_sp.at[db.at[li]], add=True)
                pltpu.sync_copy(ones_v, cnt_sp.at[db.at[li]], add=True)

                @pl.when(li + 2 < GRP)
                def _():
                    pltpu.async_copy(x_hbm.at[sb.at[li + 2]], rows[b], sems[b])
            return 0
        lax.fori_loop(0, GRP // 2, step, 0)
    plsc.subcore_barrier()

    off = c * NA + s * ROWS_PER_TILE
    pltpu.sync_copy(acc_sp.at[pl.ds(s * ROWS_PER_TILE, ROWS_PER_TILE)],
                    parts_hbm.at[pl.ds(off, ROWS_PER_TILE)])
    pltpu.sync_copy(cnt_sp.at[pl.ds(s * ROWS_PER_TILE, ROWS_PER_TILE)],
                    cnts_hbm.at[pl.ds(off, ROWS_PER_TILE)])


def _stage_a(x, src_t, dst_t):
    mesh = plsc.VectorSubcoreMesh(core_axis_name="c", subcore_axis_name="s",
                                  num_cores=NC, num_subcores=NS)
    return pl.kernel(
        _agg_body,
        out_type=(jax.ShapeDtypeStruct((NC * NA, D), _f32),
                  jax.ShapeDtypeStruct((NC * NA,), _f32)),
        mesh=mesh,
        scratch_types=[
            pltpu.VMEM_SHARED((NA, D), _f32),
            pltpu.VMEM_SHARED((NA,), _f32),
            pltpu.VMEM((GRP, CHUNK), _i32),
            pltpu.VMEM((GRP, CHUNK), _i32),
            pltpu.VMEM((GRP, CHUNK), _i32),
            pltpu.VMEM((GRP, CHUNK), _i32),
            pltpu.VMEM((CHUNK, D), _f32),
            pltpu.VMEM((CHUNK, D), _f32),
            pltpu.VMEM((CHUNK,), _f32),
            pltpu.SemaphoreType.DMA,
            pltpu.SemaphoreType.DMA,
            pltpu.SemaphoreType.DMA,
            pltpu.SemaphoreType.DMA,
        ],
        name="sage_edge_agg",
    )(x, src_t, dst_t)


def _mlp_body(p_ref, c_ref, x_ref, wl_ref, wr_ref, bl_ref, o_ref):
    p = p_ref[...]
    cm = c_ref[...]
    summed = p[0] + p[1]
    cnt = cm[0] + cm[1]
    mean = summed / jnp.maximum(cnt, 1.0)
    dn = (((1,), (1,)), ((), ()))
    h = lax.dot_general(mean, wl_ref[...], dn, preferred_element_type=_f32)
    h = h + lax.dot_general(x_ref[...], wr_ref[...], dn,
                            preferred_element_type=_f32)
    h = h + bl_ref[...]
    o_ref[...] = jnp.maximum(h, 0.0)


def _stage_b(parts, cnts, x_pad, W_l, W_r, b_l):
    RB = 512
    grid = (NA // RB,)
    return pl.pallas_call(
        _mlp_body,
        grid=grid,
        in_specs=[
            pl.BlockSpec((2, RB, D), lambda i: (0, i, 0)),
            pl.BlockSpec((2, RB, 1), lambda i: (0, i, 0)),
            pl.BlockSpec((RB, D), lambda i: (i, 0)),
            pl.BlockSpec((H, D), lambda i: (0, 0)),
            pl.BlockSpec((H, D), lambda i: (0, 0)),
            pl.BlockSpec((1, H), lambda i: (0, 0)),
        ],
        out_specs=pl.BlockSpec((RB, H), lambda i: (i, 0)),
        out_shape=jax.ShapeDtypeStruct((NA, H), _f32),
        name="sage_mlp",
    )(parts.reshape(NC, NA, D), cnts.reshape(NC, NA, 1), x_pad,
      W_l, W_r, b_l.reshape(1, H))


def _pool_body(h_hbm, batch_hbm, pooled_hbm, bbuf, cbuf, acc4, sem):
    del sem
    c = lax.axis_index("c")
    s = lax.axis_index("s")
    wid = c * NS + s
    g0 = wid * (B // (NC * NS))

    BCH = 2000
    carry = tuple(jnp.zeros((LANES,), _i32) for _ in range(5))
    for t in range(N // BCH):
        pltpu.sync_copy(batch_hbm.at[pl.ds(t * BCH, BCH)], bbuf)

        def cb(j, carry):
            v = bbuf[pl.ds(j * LANES, LANES)]
            outs = []
            for k in range(5):
                m = jnp.where(v < g0 + k, jnp.int32(1), jnp.int32(0))
                outs.append(carry[k] + m)
            return tuple(outs)
        carry = lax.fori_loop(0, BCH // LANES, cb, carry)

    bounds = []
    for k in range(5):
        vk = carry[k]
        ssum = vk[0]
        for l in range(1, LANES):
            ssum = ssum + vk[l]
        bounds.append(ssum)

    RCH = 64
    for k in range(4):
        start = bounds[k]
        end = bounds[k + 1]
        start8 = (start // 8) * 8
        nch = (end - start8 + (RCH - 1)) // RCH

        def chunk(cix, accs):
            off = pl.multiple_of(start8 + cix * RCH, 8)
            pltpu.sync_copy(h_hbm.at[pl.ds(off, RCH)], cbuf)

            def row(r, accs):
                pos = off + r
                valid = (pos >= start) & (pos < end)
                return tuple(
                    jnp.where(valid,
                              jnp.maximum(accs[j],
                                          cbuf[r, pl.ds(LANES * j, LANES)]),
                              accs[j])
                    for j in range(H // LANES))
            return lax.fori_loop(0, RCH, row, accs)

        accs = lax.fori_loop(0, nch, chunk,
                             tuple(jnp.zeros((LANES,), _f32)
                                   for _ in range(H // LANES)))
        for j in range(H // LANES):
            acc4[k, pl.ds(LANES * j, LANES)] = accs[j]

    pltpu.sync_copy(acc4, pooled_hbm.at[pl.ds(wid * 4, 4)])


def _stage_c(h, batch):
    mesh = plsc.VectorSubcoreMesh(core_axis_name="c", subcore_axis_name="s",
                                  num_cores=NC, num_subcores=NS)
    return pl.kernel(
        _pool_body,
        out_type=jax.ShapeDtypeStruct((B, H), _f32),
        mesh=mesh,
        scratch_types=[
            pltpu.VMEM((2000,), _i32),
            pltpu.VMEM((64, H), _f32),
            pltpu.VMEM((4, H), _f32),
            pltpu.SemaphoreType.DMA,
        ],
        name="sage_pool",
    )(h, batch)


def _head_body(p_ref, w4_ref, b4_ref, o_ref):
    dn = (((1,), (1,)), ((), ()))
    logits = lax.dot_general(p_ref[...], w4_ref[...], dn,
                             preferred_element_type=_f32)
    logits = logits + b4_ref[...]
    m = jnp.max(logits, axis=1, keepdims=True)
    z = logits - m
    lse = jnp.log(jnp.sum(jnp.exp(z), axis=1, keepdims=True))
    o_ref[...] = z - lse


def _stage_d(pooled, W4, b4):
    C = W4.shape[0]
    return pl.pallas_call(
        _head_body,
        out_shape=jax.ShapeDtypeStruct((B, C), _f32),
        name="sage_head",
    )(pooled, W4, b4.reshape(1, C))


@jax.jit
def kernel(x, edge_index, batch, embedding_data, W_l, b_l, W_r, W4, b4):
    del embedding_data

    pad = EPA - E
    pidx = jnp.arange(pad, dtype=_i32)
    src_p = jnp.concatenate([edge_index[0], pidx % N])
    dst_p = jnp.concatenate([edge_index[1], N + pidx % (NA - N)])
    src_t = src_p.reshape(NC, NS, NGRP, GRP, CHUNK)
    dst_t = dst_p.reshape(NC, NS, NGRP, GRP, CHUNK)

    x_pad = jnp.concatenate([x, jnp.zeros((NA - N, D), _f32)])

    parts, cnts = _stage_a(x_pad, src_t, dst_t)
    h = _stage_b(parts, cnts, x_pad, W_l, W_r, b_l)
    pooled = _stage_c(h, batch)
    return _stage_d(pooled, W4, b4)


# speedup vs baseline: 10.8073x; 1.0033x over previous
import jax
import jax.numpy as jnp
from jax import lax
from jax.experimental import pallas as pl
from jax.experimental.pallas import tpu as pltpu
from jax.experimental.pallas import tpu_sc as plsc

NC = 2
NS = 16
LANES = 16

N = 10000
E = 320000
D = 128
H = 128
B = 128

CHUNK = 128
NCH = 80
GRP = 16
NGRP = NCH // GRP
PT = NCH * CHUNK
EPA = NC * NS * PT
NA = 10240
ROWS_PER_TILE = NA // NS

_f32 = jnp.float32
_i32 = jnp.int32


def _agg_body(x_hbm, srcm_hbm, dstm_hbm, parts_hbm, cnts_hbm,
              acc_sp, cnt_sp, srci0, srci1, dsti0, dsti1, rows0, rows1,
              ones_v, gsem0, gsem1, isem0, isem1, ssem0, ssem1,
              csem0, csem1):
    c = lax.axis_index("c")
    s = lax.axis_index("s")

    def zrow(i, _):
        for j in range(D // LANES):
            rows0[i, pl.ds(LANES * j, LANES)] = jnp.zeros((LANES,), _f32)
        return 0
    lax.fori_loop(0, CHUNK, zrow, 0)

    for j in range(CHUNK // LANES):
        ones_v[pl.ds(LANES * j, LANES)] = jnp.ones((LANES,), _f32)

    for k in range(ROWS_PER_TILE // CHUNK):
        pltpu.sync_copy(rows0, acc_sp.at[pl.ds(s * ROWS_PER_TILE + k * CHUNK, CHUNK)])
        pltpu.sync_copy(rows0.at[0],
                        cnt_sp.at[pl.ds(s * ROWS_PER_TILE + k * CHUNK, CHUNK)])
    plsc.subcore_barrier()

    rows = (rows0, rows1)
    gsems = (gsem0, gsem1)
    ssems = (ssem0, ssem1)
    csems = (csem0, csem1)
    ibufs = ((srci0, dsti0, isem0), (srci1, dsti1, isem1))

    def drain_cnt(dbuf, sem):
        def body(li, _):
            pltpu.make_async_copy(ones_v, cnt_sp.at[dbuf.at[li]], sem).wait()
            return 0
        lax.fori_loop(0, GRP, body, 0)

    pltpu.sync_copy(srcm_hbm.at[c, s, 0], srci0)
    pltpu.sync_copy(dstm_hbm.at[c, s, 0], dsti0)

    for g in range(NGRP):
        sb, db, isem = ibufs[g % 2]
        if g > 0:
            pltpu.make_async_copy(srcm_hbm.at[c, s, g], sb, isem).wait()
            pltpu.make_async_copy(dstm_hbm.at[c, s, g], db, isem).wait()
            pltpu.make_async_copy(rows0, acc_sp.at[db.at[0]], ssem0).wait()
            pltpu.make_async_copy(rows1, acc_sp.at[db.at[1]], ssem1).wait()
        if g + 1 < NGRP:
            nsb, ndb, nisem = ibufs[(g + 1) % 2]
            if g >= 1:
                drain_cnt(ndb, csems[(g + 1) % 2])
            pltpu.async_copy(srcm_hbm.at[c, s, g + 1], nsb, nisem)
            pltpu.async_copy(dstm_hbm.at[c, s, g + 1], ndb, nisem)
        pltpu.async_copy(x_hbm.at[sb.at[0]], rows0, gsem0)
        pltpu.async_copy(x_hbm.at[sb.at[1]], rows1, gsem1)

        def step(i2, _):
            for b in range(2):
                li = 2 * i2 + b
                pltpu.make_async_copy(x_hbm.at[sb.at[li]], rows[b], gsems[b]).wait()
                pltpu.async_copy(rows[b], acc_sp.at[db.at[li]], ssems[b],
                                 add=True)
                pltpu.async_copy(ones_v, cnt_sp.at[db.at[li]], csems[g % 2],
                                 add=True)

                @pl.when(li + 2 < GRP)
                def _():
                    pltpu.make_async_copy(rows[b], acc_sp.at[db.at[li]],
                                          ssems[b]).wait()
                    pltpu.async_copy(x_hbm.at[sb.at[li + 2]], rows[b], gsems[b])
            return 0
        lax.fori_loop(0, GRP // 2, step, 0)

    lastdb = ibufs[(NGRP - 1) % 2][1]
    pltpu.make_async_copy(rows0, acc_sp.at[lastdb.at[GRP - 2]], ssem0).wait()
    pltpu.make_async_copy(rows1, acc_sp.at[lastdb.at[GRP - 1]], ssem1).wait()
    drain_cnt(lastdb, csems[(NGRP - 1) % 2])
    drain_cnt(ibufs[(NGRP - 2) % 2][1], csems[(NGRP - 2) % 2])
    plsc.subcore_barrier()

    off = c * NA + s * ROWS_PER_TILE
    pltpu.sync_copy(acc_sp.at[pl.ds(s * ROWS_PER_TILE, ROWS_PER_TILE)],
                    parts_hbm.at[pl.ds(off, ROWS_PER_TILE)])
    pltpu.sync_copy(cnt_sp.at[pl.ds(s * ROWS_PER_TILE, ROWS_PER_TILE)],
                    cnts_hbm.at[pl.ds(off, ROWS_PER_TILE)])


def _stage_a(x, src_t, dst_t):
    mesh = plsc.VectorSubcoreMesh(core_axis_name="c", subcore_axis_name="s",
                                  num_cores=NC, num_subcores=NS)
    return pl.kernel(
        _agg_body,
        out_type=(jax.ShapeDtypeStruct((NC * NA, D), _f32),
                  jax.ShapeDtypeStruct((NC * NA,), _f32)),
        mesh=mesh,
        scratch_types=[
            pltpu.VMEM_SHARED((NA, D), _f32),
            pltpu.VMEM_SHARED((NA,), _f32),
            pltpu.VMEM((GRP, CHUNK), _i32),
            pltpu.VMEM((GRP, CHUNK), _i32),
            pltpu.VMEM((GRP, CHUNK), _i32),
            pltpu.VMEM((GRP, CHUNK), _i32),
            pltpu.VMEM((CHUNK, D), _f32),
            pltpu.VMEM((CHUNK, D), _f32),
            pltpu.VMEM((CHUNK,), _f32),
            pltpu.SemaphoreType.DMA,
            pltpu.SemaphoreType.DMA,
            pltpu.SemaphoreType.DMA,
            pltpu.SemaphoreType.DMA,
            pltpu.SemaphoreType.DMA,
            pltpu.SemaphoreType.DMA,
            pltpu.SemaphoreType.DMA,
            pltpu.SemaphoreType.DMA,
        ],
        name="sage_edge_agg",
    )(x, src_t, dst_t)


def _mlp_body(p_ref, c_ref, x_ref, wl_ref, wr_ref, bl_ref, o_ref):
    p = p_ref[...]
    cm = c_ref[...]
    summed = p[0] + p[1]
    cnt = cm[0] + cm[1]
    mean = summed / jnp.maximum(cnt, 1.0)
    dn = (((1,), (1,)), ((), ()))
    h = lax.dot_general(mean, wl_ref[...], dn, preferred_element_type=_f32)
    h = h + lax.dot_general(x_ref[...], wr_ref[...], dn,
                            preferred_element_type=_f32)
    h = h + bl_ref[...]
    o_ref[...] = jnp.maximum(h, 0.0)


def _stage_b(parts, cnts, x, W_l, W_r, b_l):
    RB = 400
    grid = (N // RB,)
    return pl.pallas_call(
        _mlp_body,
        grid=grid,
        in_specs=[
            pl.BlockSpec((2, RB, D), lambda i: (0, i, 0)),
            pl.BlockSpec((2, RB, 1), lambda i: (0, i, 0)),
            pl.BlockSpec((RB, D), lambda i: (i, 0)),
            pl.BlockSpec((H, D), lambda i: (0, 0)),
            pl.BlockSpec((H, D), lambda i: (0, 0)),
            pl.BlockSpec((1, H), lambda i: (0, 0)),
        ],
        out_specs=pl.BlockSpec((RB, H), lambda i: (i, 0)),
        out_shape=jax.ShapeDtypeStruct((NA, H), _f32),
        name="sage_mlp",
    )(parts.reshape(NC, NA, D), cnts.reshape(NC, NA, 1), x,
      W_l, W_r, b_l.reshape(1, H))


def _pool_body(h_hbm, batch_hbm, pooled_hbm, bbuf, cbuf, acc4, sem):
    del sem
    c = lax.axis_index("c")
    s = lax.axis_index("s")
    wid = c * NS + s
    g0 = wid * (B // (NC * NS))

    BCH = 2000
    carry = tuple(jnp.zeros((LANES,), _i32) for _ in range(5))
    for t in range(N // BCH):
        pltpu.sync_copy(batch_hbm.at[pl.ds(t * BCH, BCH)], bbuf)

        def cb(j, carry):
            v = bbuf[pl.ds(j * LANES, LANES)]
            outs = []
            for k in range(5):
                m = jnp.where(v < g0 + k, jnp.int32(1), jnp.int32(0))
                outs.append(carry[k] + m)
            return tuple(outs)
        carry = lax.fori_loop(0, BCH // LANES, cb, carry)

    bounds = []
    for k in range(5):
        vk = carry[k]
        ssum = vk[0]
        for l in range(1, LANES):
            ssum = ssum + vk[l]
        bounds.append(ssum)

    RCH = 64
    for k in range(4):
        start = bounds[k]
        end = bounds[k + 1]
        start8 = (start // 8) * 8
        nch = (end - start8 + (RCH - 1)) // RCH

        def chunk(cix, accs):
            off = pl.multiple_of(start8 + cix * RCH, 8)
            pltpu.sync_copy(h_hbm.at[pl.ds(off, RCH)], cbuf)

            def row(r, accs):
                pos = off + r
                valid = (pos >= start) & (pos < end)
                return tuple(
                    jnp.where(valid,
                              jnp.maximum(accs[j],
                                          cbuf[r, pl.ds(LANES * j, LANES)]),
                              accs[j])
                    for j in range(H // LANES))
            return lax.fori_loop(0, RCH, row, accs)

        accs = lax.fori_loop(0, nch, chunk,
                             tuple(jnp.zeros((LANES,), _f32)
                                   for _ in range(H // LANES)))
        for j in range(H // LANES):
            acc4[k, pl.ds(LANES * j, LANES)] = accs[j]

    pltpu.sync_copy(acc4, pooled_hbm.at[pl.ds(wid * 4, 4)])


def _stage_c(h, batch):
    mesh = plsc.VectorSubcoreMesh(core_axis_name="c", subcore_axis_name="s",
                                  num_cores=NC, num_subcores=NS)
    return pl.kernel(
        _pool_body,
        out_type=jax.ShapeDtypeStruct((B, H), _f32),
        mesh=mesh,
        scratch_types=[
            pltpu.VMEM((2000,), _i32),
            pltpu.VMEM((64, H), _f32),
            pltpu.VMEM((4, H), _f32),
            pltpu.SemaphoreType.DMA,
        ],
        name="sage_pool",
    )(h, batch)


def _head_body(p_ref, w4_ref, b4_ref, o_ref):
    dn = (((1,), (1,)), ((), ()))
    logits = lax.dot_general(p_ref[...], w4_ref[...], dn,
                             preferred_element_type=_f32)
    logits = logits + b4_ref[...]
    m = jnp.max(logits, axis=1, keepdims=True)
    z = logits - m
    lse = jnp.log(jnp.sum(jnp.exp(z), axis=1, keepdims=True))
    o_ref[...] = z - lse


def _stage_d(pooled, W4, b4):
    C = W4.shape[0]
    return pl.pallas_call(
        _head_body,
        out_shape=jax.ShapeDtypeStruct((B, C), _f32),
        name="sage_head",
    )(pooled, W4, b4.reshape(1, C))


@jax.jit
def kernel(x, edge_index, batch, embedding_data, W_l, b_l, W_r, W4, b4):
    del embedding_data

    pad = EPA - E
    pidx = jnp.arange(pad, dtype=_i32)
    src_p = jnp.concatenate([edge_index[0], pidx % N])
    dst_p = jnp.concatenate([edge_index[1], N + pidx % (NA - N)])
    src_t = src_p.reshape(NC, NS, NGRP, GRP, CHUNK)
    dst_t = dst_p.reshape(NC, NS, NGRP, GRP, CHUNK)

    parts, cnts = _stage_a(x, src_t, dst_t)
    h = _stage_b(parts, cnts, x, W_l, W_r, b_l)
    pooled = _stage_c(h, batch)
    return _stage_d(pooled, W4, b4)


# speedup vs baseline: 11.3711x; 1.0522x over previous
import jax
import jax.numpy as jnp
from jax import lax
from jax.experimental import pallas as pl
from jax.experimental.pallas import tpu as pltpu
from jax.experimental.pallas import tpu_sc as plsc

NC = 2
NS = 16
LANES = 16

N = 10000
E = 320000
D = 128
H = 128
B = 128

CHUNK = 64
NFC = 156
NW = NC * NS
XTRA = (E - NW * NFC * CHUNK) // CHUNK
NROW = 4
NIDX = 6
NA = 10240
ROWS_PER_TILE = NA // NS

_f32 = jnp.float32
_i32 = jnp.int32


def _agg_body(x_hbm, src_hbm, dst_hbm, parts_hbm, cnts_hbm,
              acc_sp, cnt_sp,
              rows0, rows1, rows2, rows3,
              sb0, sb1, sb2, sb3, sb4, sb5,
              db0, db1, db2, db3, db4, db5,
              ones_v,
              gsem0, gsem1, gsem2, gsem3,
              ssem0, ssem1, ssem2, ssem3,
              isem0, isem1, isem2, isem3, isem4, isem5,
              csem):
    c = lax.axis_index("c")
    s = lax.axis_index("s")
    wid = c * NS + s
    ebase = wid * (NFC * CHUNK)

    rows = (rows0, rows1, rows2, rows3)
    gsems = (gsem0, gsem1, gsem2, gsem3)
    ssems = (ssem0, ssem1, ssem2, ssem3)
    sbs = (sb0, sb1, sb2, sb3, sb4, sb5)
    dbs = (db0, db1, db2, db3, db4, db5)
    isems = (isem0, isem1, isem2, isem3, isem4, isem5)

    def zrow(i, _):
        for j in range(D // LANES):
            rows0[i, pl.ds(LANES * j, LANES)] = jnp.zeros((LANES,), _f32)
        return 0
    lax.fori_loop(0, CHUNK, zrow, 0)
    for j in range(CHUNK // LANES):
        ones_v[pl.ds(LANES * j, LANES)] = jnp.ones((LANES,), _f32)

    for k in range(ROWS_PER_TILE // CHUNK):
        pltpu.sync_copy(rows0, acc_sp.at[pl.ds(s * ROWS_PER_TILE + k * CHUNK, CHUNK)])
    for k in range(ROWS_PER_TILE // D):
        pltpu.sync_copy(rows0.at[0],
                        cnt_sp.at[pl.ds(s * ROWS_PER_TILE + k * D, D)])
    plsc.subcore_barrier()

    def load_idx(ci, slot):
        off = ebase + ci * CHUNK
        pltpu.async_copy(src_hbm.at[pl.ds(off, CHUNK)], sbs[slot], isems[slot])
        pltpu.async_copy(dst_hbm.at[pl.ds(off, CHUNK)], dbs[slot], isems[slot])

    def wait_idx(ci, slot):
        off = ebase + ci * CHUNK
        pltpu.make_async_copy(src_hbm.at[pl.ds(off, CHUNK)], sbs[slot],
                              isems[slot]).wait()
        pltpu.make_async_copy(dst_hbm.at[pl.ds(off, CHUNK)], dbs[slot],
                              isems[slot]).wait()

    for k in range(4):
        load_idx(k, k)
    wait_idx(0, 0)
    pltpu.async_copy(x_hbm.at[sb0], rows0, gsem0)
    wait_idx(1, 1)
    pltpu.async_copy(x_hbm.at[sb1], rows1, gsem1)

    def step(i12, _):
        for u in range(12):
            r4 = u % NROW
            r6 = u % NIDX
            ci = i12 * 12 + u
            pltpu.make_async_copy(x_hbm.at[sbs[r6]], rows[r4], gsems[r4]).wait()
            pltpu.async_copy(rows[r4], acc_sp.at[dbs[r6]], ssems[r4], add=True)
            pltpu.async_copy(ones_v, cnt_sp.at[dbs[r6]], csem, add=True)

            @pl.when(ci >= 2)
            def _():
                pltpu.make_async_copy(ones_v, cnt_sp.at[dbs[r6]], csem).wait()
                pltpu.make_async_copy(rows[(u + 2) % NROW],
                                      acc_sp.at[dbs[r6]],
                                      ssems[(u + 2) % NROW]).wait()

            @pl.when(ci + 2 < NFC)
            def _():
                wait_idx(ci + 2, (u + 2) % NIDX)
                pltpu.async_copy(x_hbm.at[sbs[(u + 2) % NIDX]],
                                 rows[(u + 2) % NROW], gsems[(u + 2) % NROW])

            @pl.when(ci + 4 < NFC)
            def _():
                load_idx(ci + 4, (u + 4) % NIDX)
        return 0
    lax.fori_loop(0, NFC // 12, step, 0)

    for ci in (NFC - 2, NFC - 1):
        r4 = ci % NROW
        r6 = ci % NIDX
        pltpu.make_async_copy(rows[r4], acc_sp.at[dbs[r6]], ssems[r4]).wait()
        pltpu.make_async_copy(ones_v, cnt_sp.at[dbs[r6]], csem).wait()

    @pl.when(wid < XTRA)
    def _():
        off = NW * NFC * CHUNK + wid * CHUNK
        pltpu.sync_copy(src_hbm.at[pl.ds(off, CHUNK)], sb0)
        pltpu.sync_copy(dst_hbm.at[pl.ds(off, CHUNK)], db0)
        pltpu.async_copy(x_hbm.at[sb0], rows0, gsem0).wait()
        pltpu.sync_copy(rows0, acc_sp.at[db0], add=True)
        pltpu.sync_copy(ones_v, cnt_sp.at[db0], add=True)

    plsc.subcore_barrier()

    pltpu.sync_copy(acc_sp.at[pl.ds(s * ROWS_PER_TILE, ROWS_PER_TILE)],
                    parts_hbm.at[c, pl.ds(s * ROWS_PER_TILE, ROWS_PER_TILE)])
    pltpu.sync_copy(cnt_sp.at[pl.ds(s * ROWS_PER_TILE, ROWS_PER_TILE)],
                    cnts_hbm.at[pl.ds(c * NA + s * ROWS_PER_TILE,
                                      ROWS_PER_TILE)])


def _stage_a(x, src, dst):
    mesh = plsc.VectorSubcoreMesh(core_axis_name="c", subcore_axis_name="s",
                                  num_cores=NC, num_subcores=NS)
    idx = [pltpu.VMEM((CHUNK,), _i32) for _ in range(2 * NIDX)]
    return pl.kernel(
        _agg_body,
        out_type=(jax.ShapeDtypeStruct((NC, NA, D), _f32),
                  jax.ShapeDtypeStruct((NC * NA,), _f32)),
        mesh=mesh,
        scratch_types=(
            [pltpu.VMEM_SHARED((NA, D), _f32),
             pltpu.VMEM_SHARED((NA,), _f32)]
            + [pltpu.VMEM((CHUNK, D), _f32) for _ in range(NROW)]
            + idx
            + [pltpu.VMEM((CHUNK,), _f32)]
            + [pltpu.SemaphoreType.DMA] * (NROW + NROW + NIDX + 1)
        ),
        name="sage_edge_agg",
    )(x, src, dst)


def _mlp_body(p_ref, c_ref, x_ref, wl_ref, wr_ref, bl_ref, o_ref):
    p = p_ref[...]
    cm = c_ref[...]
    summed = p[0] + p[1]
    invc = 1.0 / jnp.maximum(cm[0, 0] + cm[1, 0], 1.0)
    dn_t = (((1,), (1,)), ((), ()))
    dn_n = (((1,), (0,)), ((), ()))
    ii = jnp.equal(lax.broadcasted_iota(_i32, (128, 128), 0),
                   lax.broadcasted_iota(_i32, (128, 128), 1))
    means = []
    nq = summed.shape[0] // 128
    for q in range(nq):
        dq = jnp.where(ii, invc[q][None, :], 0.0)
        sq = summed[q * 128:(q + 1) * 128]
        means.append(lax.dot_general(dq, sq, dn_n, preferred_element_type=_f32))
    mean = jnp.concatenate(means, axis=0)
    h = lax.dot_general(mean, wl_ref[...], dn_t, preferred_element_type=_f32)
    h = h + lax.dot_general(x_ref[...], wr_ref[...], dn_t,
                            preferred_element_type=_f32)
    h = h + bl_ref[...]
    o_ref[...] = jnp.maximum(h, 0.0)


def _stage_b(parts, cnts, x, W_l, W_r, b_l):
    RB = 512
    grid = (NA // RB,)
    return pl.pallas_call(
        _mlp_body,
        grid=grid,
        in_specs=[
            pl.BlockSpec((2, RB, D), lambda i: (0, i, 0)),
            pl.BlockSpec((2, 1, RB // 128, 128), lambda i: (0, i, 0, 0)),
            pl.BlockSpec((RB, D), lambda i: (i, 0)),
            pl.BlockSpec((H, D), lambda i: (0, 0)),
            pl.BlockSpec((H, D), lambda i: (0, 0)),
            pl.BlockSpec((1, H), lambda i: (0, 0)),
        ],
        out_specs=pl.BlockSpec((RB, H), lambda i: (i, 0)),
        out_shape=jax.ShapeDtypeStruct((NA, H), _f32),
        name="sage_mlp",
    )(parts, cnts.reshape(NC, NA // 512, 4, 128), x, W_l, W_r,
      b_l.reshape(1, H))


def _pool_body(h_hbm, batch_hbm, pooled_hbm, bbuf, cbuf, acc4, sem):
    del sem
    c = lax.axis_index("c")
    s = lax.axis_index("s")
    wid = c * NS + s
    g0 = wid * (B // (NC * NS))

    BCH = 2000
    carry = tuple(jnp.zeros((LANES,), _i32) for _ in range(5))
    for t in range(N // BCH):
        pltpu.sync_copy(batch_hbm.at[pl.ds(t * BCH, BCH)], bbuf)

        def cb(j, carry):
            v = bbuf[pl.ds(j * LANES, LANES)]
            outs = []
            for k in range(5):
                m = jnp.where(v < g0 + k, jnp.int32(1), jnp.int32(0))
                outs.append(carry[k] + m)
            return tuple(outs)
        carry = lax.fori_loop(0, BCH // LANES, cb, carry)

    bounds = []
    for k in range(5):
        vk = carry[k]
        ssum = vk[0]
        for l in range(1, LANES):
            ssum = ssum + vk[l]
        bounds.append(ssum)

    RCH = 64
    for k in range(4):
        start = bounds[k]
        end = bounds[k + 1]
        start8 = (start // 8) * 8
        nch = (end - start8 + (RCH - 1)) // RCH

        def chunk(cix, accs):
            off = pl.multiple_of(start8 + cix * RCH, 8)
            pltpu.sync_copy(h_hbm.at[pl.ds(off, RCH)], cbuf)

            def row(r, accs):
                pos = off + r
                valid = (pos >= start) & (pos < end)
                return tuple(
                    jnp.where(valid,
                              jnp.maximum(accs[j],
                                          cbuf[r, pl.ds(LANES * j, LANES)]),
                              accs[j])
                    for j in range(H // LANES))
            return lax.fori_loop(0, RCH, row, accs)

        accs = lax.fori_loop(0, nch, chunk,
                             tuple(jnp.zeros((LANES,), _f32)
                                   for _ in range(H // LANES)))
        for j in range(H // LANES):
            acc4[k, pl.ds(LANES * j, LANES)] = accs[j]

    pltpu.sync_copy(acc4, pooled_hbm.at[pl.ds(wid * 4, 4)])


def _stage_c(h, batch):
    mesh = plsc.VectorSubcoreMesh(core_axis_name="c", subcore_axis_name="s",
                                  num_cores=NC, num_subcores=NS)
    return pl.kernel(
        _pool_body,
        out_type=jax.ShapeDtypeStruct((B, H), _f32),
        mesh=mesh,
        scratch_types=[
            pltpu.VMEM((2000,), _i32),
            pltpu.VMEM((64, H), _f32),
            pltpu.VMEM((4, H), _f32),
            pltpu.SemaphoreType.DMA,
        ],
        name="sage_pool",
    )(h, batch)


def _head_body(p_ref, w4_ref, b4_ref, o_ref):
    dn = (((1,), (1,)), ((), ()))
    logits = lax.dot_general(p_ref[...], w4_ref[...], dn,
                             preferred_element_type=_f32)
    logits = logits + b4_ref[...]
    m = jnp.max(logits, axis=1, keepdims=True)
    z = logits - m
    lse = jnp.log(jnp.sum(jnp.exp(z), axis=1, keepdims=True))
    o_ref[...] = z - lse


def _stage_d(pooled, W4, b4):
    C = W4.shape[0]
    return pl.pallas_call(
        _head_body,
        out_shape=jax.ShapeDtypeStruct((B, C), _f32),
        name="sage_head",
    )(pooled, W4, b4.reshape(1, C))


@jax.jit
def kernel(x, edge_index, batch, embedding_data, W_l, b_l, W_r, W4, b4):
    del embedding_data

    src = edge_index[0]
    dst = edge_index[1]

    parts, cnts = _stage_a(x, src, dst)
    h = _stage_b(parts, cnts, x, W_l, W_r, b_l)
    pooled = _stage_c(h, batch)
    return _stage_d(pooled, W4, b4)


# speedup vs baseline: 11.7989x; 1.0376x over previous
import jax
import jax.numpy as jnp
from jax import lax
from jax.experimental import pallas as pl
from jax.experimental.pallas import tpu as pltpu
from jax.experimental.pallas import tpu_sc as plsc

NC = 2
NS = 16
LANES = 16

N = 10000
E = 320000
D = 128
H = 128
B = 128

CHUNK = 32
NFC = 312
NW = NC * NS
XTRA = (E - NW * NFC * CHUNK) // CHUNK
NROW = 8
NIDX = 12
UNROLL = 24
NA = 10240
ROWS_PER_TILE = NA // NS

_f32 = jnp.float32
_i32 = jnp.int32


def _agg_body(x_hbm, src_hbm, dst_hbm, parts_hbm, cnts_hbm,
              acc_sp, cnt_sp,
              rows0, rows1, rows2, rows3, rows4, rows5, rows6, rows7,
              sb0, sb1, sb2, sb3, sb4, sb5,
              sb6, sb7, sb8, sb9, sb10, sb11,
              db0, db1, db2, db3, db4, db5,
              db6, db7, db8, db9, db10, db11,
              ones_v,
              gsem0, gsem1, gsem2, gsem3, gsem4, gsem5, gsem6, gsem7,
              ssem0, ssem1, ssem2, ssem3, ssem4, ssem5, ssem6, ssem7,
              isem0, isem1, isem2, isem3, isem4, isem5,
              isem6, isem7, isem8, isem9, isem10, isem11,
              csem):
    c = lax.axis_index("c")
    s = lax.axis_index("s")
    wid = c * NS + s
    ebase = wid * (NFC * CHUNK)

    rows = (rows0, rows1, rows2, rows3, rows4, rows5, rows6, rows7)
    gsems = (gsem0, gsem1, gsem2, gsem3, gsem4, gsem5, gsem6, gsem7)
    ssems = (ssem0, ssem1, ssem2, ssem3, ssem4, ssem5, ssem6, ssem7)
    sbs = (sb0, sb1, sb2, sb3, sb4, sb5, sb6, sb7, sb8, sb9, sb10, sb11)
    dbs = (db0, db1, db2, db3, db4, db5, db6, db7, db8, db9, db10, db11)
    isems = (isem0, isem1, isem2, isem3, isem4, isem5,
             isem6, isem7, isem8, isem9, isem10, isem11)

    def zrow(i, _):
        for j in range(D // LANES):
            rows0[i, pl.ds(LANES * j, LANES)] = jnp.zeros((LANES,), _f32)
        return 0
    lax.fori_loop(0, CHUNK, zrow, 0)
    for j in range(CHUNK // LANES):
        ones_v[pl.ds(LANES * j, LANES)] = jnp.ones((LANES,), _f32)

    for k in range(ROWS_PER_TILE // CHUNK):
        pltpu.sync_copy(rows0, acc_sp.at[pl.ds(s * ROWS_PER_TILE + k * CHUNK, CHUNK)])
    for k in range(ROWS_PER_TILE // D):
        pltpu.sync_copy(rows0.at[0],
                        cnt_sp.at[pl.ds(s * ROWS_PER_TILE + k * D, D)])
    plsc.subcore_barrier()

    def load_idx(ci, slot):
        off = ebase + ci * CHUNK
        pltpu.async_copy(src_hbm.at[pl.ds(off, CHUNK)], sbs[slot], isems[slot])
        pltpu.async_copy(dst_hbm.at[pl.ds(off, CHUNK)], dbs[slot], isems[slot])

    def wait_idx(ci, slot):
        off = ebase + ci * CHUNK
        pltpu.make_async_copy(src_hbm.at[pl.ds(off, CHUNK)], sbs[slot],
                              isems[slot]).wait()
        pltpu.make_async_copy(dst_hbm.at[pl.ds(off, CHUNK)], dbs[slot],
                              isems[slot]).wait()

    def fire_gather(rs, isl):
        pltpu.async_copy(x_hbm.at[sbs[isl]], rows[rs], gsems[rs])

    def wait_gather(rs, isl):
        pltpu.make_async_copy(x_hbm.at[sbs[isl]], rows[rs],
                              gsems[rs]).wait()

    def fire_scatter(rs, isl):
        pltpu.async_copy(rows[rs], acc_sp.at[dbs[isl]], ssems[rs], add=True)
        pltpu.async_copy(ones_v, cnt_sp.at[dbs[isl]], csem, add=True)

    def wait_scatter(rs, isl):
        pltpu.make_async_copy(rows[rs], acc_sp.at[dbs[isl]],
                              ssems[rs]).wait()
        pltpu.make_async_copy(ones_v, cnt_sp.at[dbs[isl]], csem).wait()

    for k in range(8):
        load_idx(k, k)
    for k in range(4):
        wait_idx(k, k)
        fire_gather(k % NROW, k % NIDX)

    def step(iu, _):
        for u in range(UNROLL):
            ci = iu * UNROLL + u
            wait_gather(u % NROW, u % NIDX)
            fire_scatter(u % NROW, u % NIDX)

            @pl.when(ci >= 4)
            def _():
                wait_scatter((u - 4) % NROW, (u - 4) % NIDX)

            @pl.when(ci + 8 < NFC)
            def _():
                load_idx(ci + 8, (u + 8) % NIDX)

            @pl.when(ci + 4 < NFC)
            def _():
                wait_idx(ci + 4, (u + 4) % NIDX)
                fire_gather((u + 4) % NROW, (u + 4) % NIDX)
        return 0
    lax.fori_loop(0, NFC // UNROLL, step, 0)

    for ci in range(NFC - 4, NFC):
        wait_scatter(ci % NROW, ci % NIDX)

    @pl.when(wid < XTRA)
    def _():
        off = NW * NFC * CHUNK + wid * CHUNK
        pltpu.sync_copy(src_hbm.at[pl.ds(off, CHUNK)], sb0)
        pltpu.sync_copy(dst_hbm.at[pl.ds(off, CHUNK)], db0)
        pltpu.async_copy(x_hbm.at[sb0], rows0, gsem0).wait()
        pltpu.sync_copy(rows0, acc_sp.at[db0], add=True)
        pltpu.sync_copy(ones_v, cnt_sp.at[db0], add=True)

    plsc.subcore_barrier()

    pltpu.sync_copy(acc_sp.at[pl.ds(s * ROWS_PER_TILE, ROWS_PER_TILE)],
                    parts_hbm.at[c, pl.ds(s * ROWS_PER_TILE, ROWS_PER_TILE)])
    pltpu.sync_copy(cnt_sp.at[pl.ds(s * ROWS_PER_TILE, ROWS_PER_TILE)],
                    cnts_hbm.at[pl.ds(c * NA + s * ROWS_PER_TILE,
                                      ROWS_PER_TILE)])


def _stage_a(x, src, dst):
    mesh = plsc.VectorSubcoreMesh(core_axis_name="c", subcore_axis_name="s",
                                  num_cores=NC, num_subcores=NS)
    idx = [pltpu.VMEM((CHUNK,), _i32) for _ in range(2 * NIDX)]
    return pl.kernel(
        _agg_body,
        out_type=(jax.ShapeDtypeStruct((NC, NA, D), _f32),
                  jax.ShapeDtypeStruct((NC * NA,), _f32)),
        mesh=mesh,
        scratch_types=(
            [pltpu.VMEM_SHARED((NA, D), _f32),
             pltpu.VMEM_SHARED((NA,), _f32)]
            + [pltpu.VMEM((CHUNK, D), _f32) for _ in range(NROW)]
            + idx
            + [pltpu.VMEM((CHUNK,), _f32)]
            + [pltpu.SemaphoreType.DMA] * (NROW + NROW + NIDX + 1)
        ),
        name="sage_edge_agg",
    )(x, src, dst)


def _mlp_body(p_ref, c_ref, x_ref, wl_ref, wr_ref, bl_ref, o_ref):
    p = p_ref[...]
    cm = c_ref[...]
    summed = p[0] + p[1]
    invc = 1.0 / jnp.maximum(cm[0, 0] + cm[1, 0], 1.0)
    dn_t = (((1,), (1,)), ((), ()))
    dn_n = (((1,), (0,)), ((), ()))
    ii = jnp.equal(lax.broadcasted_iota(_i32, (128, 128), 0),
                   lax.broadcasted_iota(_i32, (128, 128), 1))
    means = []
    nq = summed.shape[0] // 128
    for q in range(nq):
        dq = jnp.where(ii, invc[q][None, :], 0.0)
        sq = summed[q * 128:(q + 1) * 128]
        means.append(lax.dot_general(dq, sq, dn_n, preferred_element_type=_f32))
    mean = jnp.concatenate(means, axis=0)
    h = lax.dot_general(mean, wl_ref[...], dn_t, preferred_element_type=_f32)
    h = h + lax.dot_general(x_ref[...], wr_ref[...], dn_t,
                            preferred_element_type=_f32)
    h = h + bl_ref[...]
    o_ref[...] = jnp.maximum(h, 0.0)


def _stage_b(parts, cnts, x, W_l, W_r, b_l):
    RB = 512
    grid = (NA // RB,)
    return pl.pallas_call(
        _mlp_body,
        grid=grid,
        in_specs=[
            pl.BlockSpec((2, RB, D), lambda i: (0, i, 0)),
            pl.BlockSpec((2, 1, RB // 128, 128), lambda i: (0, i, 0, 0)),
            pl.BlockSpec((RB, D), lambda i: (i, 0)),
            pl.BlockSpec((H, D), lambda i: (0, 0)),
            pl.BlockSpec((H, D), lambda i: (0, 0)),
            pl.BlockSpec((1, H), lambda i: (0, 0)),
        ],
        out_specs=pl.BlockSpec((RB, H), lambda i: (i, 0)),
        out_shape=jax.ShapeDtypeStruct((NA, H), _f32),
        name="sage_mlp",
    )(parts, cnts.reshape(NC, NA // 512, 4, 128), x, W_l, W_r,
      b_l.reshape(1, H))


def _pool_body(h_hbm, batch_hbm, pooled_hbm, bbuf, cbuf, acc4, sem):
    del sem
    c = lax.axis_index("c")
    s = lax.axis_index("s")
    wid = c * NS + s
    g0 = wid * (B // (NC * NS))

    BCH = 2000
    carry = tuple(jnp.zeros((LANES,), _i32) for _ in range(5))
    for t in range(N // BCH):
        pltpu.sync_copy(batch_hbm.at[pl.ds(t * BCH, BCH)], bbuf)

        def cb(j, carry):
            v = bbuf[pl.ds(j * LANES, LANES)]
            outs = []
            for k in range(5):
                m = jnp.where(v < g0 + k, jnp.int32(1), jnp.int32(0))
                outs.append(carry[k] + m)
            return tuple(outs)
        carry = lax.fori_loop(0, BCH // LANES, cb, carry)

    bounds = []
    for k in range(5):
        vk = carry[k]
        ssum = vk[0]
        for l in range(1, LANES):
            ssum = ssum + vk[l]
        bounds.append(ssum)

    RCH = 64
    for k in range(4):
        start = bounds[k]
        end = bounds[k + 1]
        start8 = (start // 8) * 8
        nch = (end - start8 + (RCH - 1)) // RCH

        def chunk(cix, accs):
            off = pl.multiple_of(start8 + cix * RCH, 8)
            pltpu.sync_copy(h_hbm.at[pl.ds(off, RCH)], cbuf)

            def row(r, accs):
                pos = off + r
                valid = (pos >= start) & (pos < end)
                return tuple(
                    jnp.where(valid,
                              jnp.maximum(accs[j],
                                          cbuf[r, pl.ds(LANES * j, LANES)]),
                              accs[j])
                    for j in range(H // LANES))
            return lax.fori_loop(0, RCH, row, accs)

        accs = lax.fori_loop(0, nch, chunk,
                             tuple(jnp.zeros((LANES,), _f32)
                                   for _ in range(H // LANES)))
        for j in range(H // LANES):
            acc4[k, pl.ds(LANES * j, LANES)] = accs[j]

    pltpu.sync_copy(acc4, pooled_hbm.at[pl.ds(wid * 4, 4)])


def _stage_c(h, batch):
    mesh = plsc.VectorSubcoreMesh(core_axis_name="c", subcore_axis_name="s",
                                  num_cores=NC, num_subcores=NS)
    return pl.kernel(
        _pool_body,
        out_type=jax.ShapeDtypeStruct((B, H), _f32),
        mesh=mesh,
        scratch_types=[
            pltpu.VMEM((2000,), _i32),
            pltpu.VMEM((64, H), _f32),
            pltpu.VMEM((4, H), _f32),
            pltpu.SemaphoreType.DMA,
        ],
        name="sage_pool",
    )(h, batch)


def _head_body(p_ref, w4_ref, b4_ref, o_ref):
    dn = (((1,), (1,)), ((), ()))
    logits = lax.dot_general(p_ref[...], w4_ref[...], dn,
                             preferred_element_type=_f32)
    logits = logits + b4_ref[...]
    m = jnp.max(logits, axis=1, keepdims=True)
    z = logits - m
    lse = jnp.log(jnp.sum(jnp.exp(z), axis=1, keepdims=True))
    o_ref[...] = z - lse


def _stage_d(pooled, W4, b4):
    C = W4.shape[0]
    return pl.pallas_call(
        _head_body,
        out_shape=jax.ShapeDtypeStruct((B, C), _f32),
        name="sage_head",
    )(pooled, W4, b4.reshape(1, C))


@jax.jit
def kernel(x, edge_index, batch, embedding_data, W_l, b_l, W_r, W4, b4):
    del embedding_data

    src = edge_index[0]
    dst = edge_index[1]

    parts, cnts = _stage_a(x, src, dst)
    h = _stage_b(parts, cnts, x, W_l, W_r, b_l)
    pooled = _stage_c(h, batch)
    return _stage_d(pooled, W4, b4)
